# baseline (speedup 1.0000x reference)
import functools

import jax
import jax.numpy as jnp
from jax import lax
from jax.experimental import pallas as pl
from jax.experimental.pallas import tpu as pltpu
from jax.experimental.pallas import tpu_sc as plsc

NUM_LODS = 10
TEX_H = 512
TEX_W = 512
NUM_CHANNELS = 11
BATCH = 1048576

CPAD = 16

NC = 2
NS = 16
L = 16
NW = NC * NS

BPW = BATCH // NW
CHUNK = 4096
NCHUNK = BPW // CHUNK

_mesh = plsc.VectorSubcoreMesh(core_axis_name="c", subcore_axis_name="s")


@functools.partial(
    pl.kernel,
    out_type=jax.ShapeDtypeStruct((BATCH, CPAD), jnp.float32),
    mesh=_mesh,
    compiler_params=pltpu.CompilerParams(use_tc_tiling_on_sc=False),
    scratch_types=[
        pltpu.VMEM((CHUNK,), jnp.int32),
        pltpu.VMEM((CHUNK,), jnp.int32),
        pltpu.VMEM((CHUNK,), jnp.int32),
        pltpu.VMEM((CHUNK,), jnp.int32),
        pltpu.VMEM((CHUNK, CPAD), jnp.float32),
        pltpu.SemaphoreType.DMA,
    ],
)
def _tex_gather(table, ys, xs, lods, out, ys_v, xs_v, lods_v, idx_v, rows_v, sem):
    wid = lax.axis_index("s") * NC + lax.axis_index("c")
    base = wid * BPW

    def chunk_body(ci, _):
        cbase = base + ci * CHUNK
        pltpu.sync_copy(ys.at[pl.ds(cbase, CHUNK)], ys_v)
        pltpu.sync_copy(xs.at[pl.ds(cbase, CHUNK)], xs_v)
        pltpu.sync_copy(lods.at[pl.ds(cbase, CHUNK)], lods_v)

        def vec_body(vi, _):
            s = pl.ds(vi * L, L)
            y = ys_v[s]
            x = xs_v[s]
            ld = lods_v[s]
            idx = (
                ld * (TEX_H * TEX_W)
                + lax.shift_right_logical(y, ld) * TEX_W
                + lax.shift_right_logical(x, ld)
            )
            idx_v[pl.ds(vi * L, L)] = idx
            return 0

        lax.fori_loop(0, CHUNK // L, vec_body, 0, unroll=4)
        pltpu.async_copy(table.at[idx_v], rows_v, sem).wait()
        pltpu.sync_copy(rows_v, out.at[pl.ds(cbase, CHUNK)])
        return 0

    lax.fori_loop(0, NCHUNK, chunk_body, 0)


def kernel(lod_cache, batch_index):
    table = lod_cache.reshape(NUM_LODS * TEX_H * TEX_W, NUM_CHANNELS)
    table = jnp.pad(table, ((0, 0), (0, CPAD - NUM_CHANNELS)))
    bi = batch_index.astype(jnp.int32)
    out = _tex_gather(table, bi[:, 0], bi[:, 1], bi[:, 2])
    return out[:, :NUM_CHANNELS]


# speedup vs baseline: 1.2729x; 1.2729x over previous
import functools

import jax
import jax.numpy as jnp
from jax import lax
from jax.experimental import pallas as pl
from jax.experimental.pallas import tpu as pltpu
from jax.experimental.pallas import tpu_sc as plsc

NUM_LODS = 10
TEX_H = 512
TEX_W = 512
NCH = 11
BATCH = 1048576

CPAD = 16
TSLAB = 1 << 19

NC = 2
NS = 16
L = 16
NW = NC * NS

BPW = BATCH // NW
CHUNK = 2048
NCHUNK = BPW // CHUNK

_PLANE = TEX_H * TEX_W * NCH

_mesh = plsc.VectorSubcoreMesh(core_axis_name="c", subcore_axis_name="s")


def _iota():
    return lax.iota(jnp.int32, L)


_GDN = lax.GatherDimensionNumbers(
    offset_dims=(), collapsed_slice_dims=(0,), start_index_map=(0,))


def _rot(v, sh):
    idx = (_iota() + (sh % L)) & (L - 1)
    return lax.gather(v, idx[:, None], _GDN, (1,),
                      mode=lax.GatherScatterMode.PROMISE_IN_BOUNDS)


@functools.partial(
    pl.kernel,
    out_type=jax.ShapeDtypeStruct((BATCH * NCH,), jnp.float32),
    mesh=_mesh,
    compiler_params=pltpu.CompilerParams(use_tc_tiling_on_sc=False),
    scratch_types=[
        pltpu.HBM((NC * TSLAB, CPAD), jnp.float32),
        pltpu.VMEM((TEX_W * NCH,), jnp.float32),
        pltpu.VMEM((TEX_W, CPAD), jnp.float32),
        pltpu.VMEM((CHUNK,), jnp.int32),
        pltpu.VMEM((CHUNK,), jnp.int32),
        pltpu.VMEM((CHUNK,), jnp.int32),
        pltpu.VMEM((CHUNK,), jnp.int32),
        pltpu.VMEM((CHUNK, CPAD), jnp.float32),
        pltpu.VMEM((CHUNK * NCH,), jnp.float32),
        pltpu.SemaphoreType.DMA,
    ],
)
def _tex_gather(cache, ys, xs, lods, out, table, stage_in, stage_out,
                ys_v, xs_v, lods_v, idx_v, rows_v, pack_v, sem):
    cid = lax.axis_index("c")
    sid = lax.axis_index("s")
    wid = sid * NC + cid
    slab = cid * TSLAB

    def expand_group(g, wk):
        a = [stage_in[pl.ds(g * (L * NCH) + L * v, L)] for v in range(NCH)]
        for jj in range(min(L, wk)):
            off = (NCH * jj) % L
            v0 = (NCH * jj) // L
            r = _rot(a[v0], off)
            if off > L - NCH:
                r = jnp.where(_iota() < (L - off), r, _rot(a[v0 + 1], off))
            stage_out[g * L + jj, :] = r

    for k in range(NUM_LODS):
        wk = TEX_W >> k
        base_k = TSLAB - (TSLAB >> k)
        rlen = -(-(wk * NCH) // 8) * 8

        def rep_one(y, k=k, wk=wk, base_k=base_k, rlen=rlen):
            pltpu.sync_copy(
                cache.at[pl.ds(k * _PLANE + y * (TEX_W * NCH), rlen)],
                stage_in.at[pl.ds(0, rlen)],
            )
            if wk >= L:
                def grp(g, _):
                    expand_group(g, wk)
                    return 0
                lax.fori_loop(0, wk // L, grp, 0)
            else:
                expand_group(0, wk)
            pltpu.sync_copy(
                stage_out.at[pl.ds(0, wk)],
                table.at[pl.ds(slab + base_k + y * wk, wk)],
            )

        if wk >= NS:
            rows_per = wk // NS

            def rep_body(r, _, rows_per=rows_per, rep_one=rep_one):
                rep_one(sid * rows_per + r)
                return 0

            lax.fori_loop(0, rows_per, rep_body, 0)
        else:
            @pl.when(sid < wk)
            def _(rep_one=rep_one):
                rep_one(sid)

    plsc.subcore_barrier()

    base = wid * BPW

    def chunk_body(ci, _):
        cbase = base + ci * CHUNK
        pltpu.sync_copy(ys.at[pl.ds(cbase, CHUNK)], ys_v)
        pltpu.sync_copy(xs.at[pl.ds(cbase, CHUNK)], xs_v)
        pltpu.sync_copy(lods.at[pl.ds(cbase, CHUNK)], lods_v)

        def vec_body(vi, _):
            s = pl.ds(vi * L, L)
            y = ys_v[s]
            x = xs_v[s]
            ld = lods_v[s]
            lbase = TSLAB - lax.shift_right_logical(
                jnp.full((L,), TSLAB, jnp.int32), ld)
            w = lax.shift_right_logical(jnp.full((L,), TEX_W, jnp.int32), ld)
            idx = (
                slab + lbase
                + lax.shift_right_logical(y, ld) * w
                + lax.shift_right_logical(x, ld)
            )
            idx_v[s] = idx
            return 0

        lax.fori_loop(0, CHUNK // L, vec_body, 0, unroll=4)
        pltpu.async_copy(table.at[idx_v], rows_v, sem).wait()

        def press_body(g, _):
            t_r = [rows_v[g * L + t, :] for t in range(L)]
            for m in range(NCH):
                t0 = (L * m) // NCH
                acc = _rot(t_r[t0], L * m - NCH * t0)
                for d in (1, 2):
                    s_d = NCH * (t0 + d) - L * m
                    if s_d < L:
                        acc = jnp.where(
                            _iota() < s_d, acc,
                            _rot(t_r[t0 + d], L * m - NCH * (t0 + d)))
                pack_v[pl.ds(g * (L * NCH) + L * m, L)] = acc
            return 0

        lax.fori_loop(0, CHUNK // L, press_body, 0)
        pltpu.sync_copy(pack_v, out.at[pl.ds(cbase * NCH, CHUNK * NCH)])
        return 0

    lax.fori_loop(0, NCHUNK, chunk_body, 0)


def kernel(lod_cache, batch_index):
    bi = batch_index.astype(jnp.int32)
    flat = _tex_gather(lod_cache.reshape(-1), bi[:, 0], bi[:, 1], bi[:, 2])
    return flat.reshape(BATCH, NCH)
